# baseline (speedup 1.0000x reference)
import functools

import jax
import jax.numpy as jnp
from jax import lax
from jax.experimental import pallas as pl
from jax.experimental.pallas import tpu as pltpu
from jax.experimental.pallas import tpu_sc as plsc

B, H, W, C = 4, 192, 192, 96
OUT_H, OUT_W = 2 * H, 2 * W
NUPD = H * W * C
POUT = OUT_H * OUT_W * C
NREG = 8
RSZ = POUT // NREG
PAD = 16
NSUB = 16
PER_TILE = NUPD // NSUB
CH = 2304
NCH = PER_TILE // CH
ZPT = RSZ // NSUB
NZ = ZPT // CH
NG = CH // 96
SCAT = CH
PCH = SCAT + 112

_mesh = plsc.VectorSubcoreMesh(core_axis_name="c", subcore_axis_name="s")


@functools.partial(
    pl.kernel,
    mesh=_mesh,
    out_type=jax.ShapeDtypeStruct((B * POUT,), jnp.float32),
    compiler_params=pltpu.CompilerParams(needs_layout_passes=False),
    scratch_types=[
        [pltpu.VMEM((CH,), jnp.int32)] * 2,
        [pltpu.VMEM((CH,), jnp.float32)] * 2,
        pltpu.VMEM((PCH,), jnp.int32),
        pltpu.VMEM((PCH,), jnp.float32),
        pltpu.VMEM((16,), jnp.int32),
        pltpu.VMEM((CH,), jnp.float32),
        pltpu.VMEM_SHARED((RSZ + PAD,), jnp.float32),
        [pltpu.SemaphoreType.DMA] * 2,
        [pltpu.SemaphoreType.DMA] * 2,
        pltpu.SemaphoreType.DMA,
        pltpu.SemaphoreType.DMA,
    ],
)
def _unpool(feat_hbm, idx_hbm, out_hbm, idx_v, feat_v, pend_off, pend_val,
            pc_v, zero_v, acc, sem_li, sem_lf, sem_z, sem_w):
    core = lax.axis_index("c")
    sub = lax.axis_index("s")
    third = jnp.float32(1.0) / jnp.float32(3.0)
    iota = lax.iota(jnp.int32, 16)

    def zfill(i, carry):
        zero_v[pl.ds(i * 16, 16)] = jnp.zeros((16,), jnp.float32)
        return carry

    lax.fori_loop(0, CH // 16, zfill, 0)

    def pfill(i, carry):
        pend_off[pl.ds(i * 16, 16)] = jnp.zeros((16,), jnp.int32)
        pend_val[pl.ds(i * 16, 16)] = jnp.zeros((16,), jnp.float32)
        return carry

    lax.fori_loop(0, PCH // 16, pfill, 0)

    def flush_body():
        pltpu.sync_copy(pend_val, acc.at[pend_off], add=True)

        def vz(i, carry):
            pend_val[pl.ds(i * 16, 16)] = jnp.zeros((16,), jnp.float32)
            return carry

        lax.fori_loop(0, PCH // 16, vz, 0)

    def maybe_flush(cnt):
        do_flush = cnt >= SCAT

        @pl.when(do_flush)
        def _():
            flush_body()

        return jnp.where(do_flush, 0, cnt)

    def one_pass(b, rr):
        rbase = (core * (NREG // 2) + rr) * RSZ
        base_in = b * NUPD + sub * PER_TILE
        cvec = [iota + (16 * k - rbase) for k in range(6)]

        def load(g, slot):
            pltpu.async_copy(
                idx_hbm.at[pl.ds(base_in + g * CH, CH)], idx_v[slot],
                sem_li[slot])
            pltpu.async_copy(
                feat_hbm.at[pl.ds(base_in + g * CH, CH)], feat_v[slot],
                sem_lf[slot])

        load(0, 0)

        def zero_issue(k, carry):
            pltpu.async_copy(zero_v, acc.at[pl.ds(sub * ZPT + k * CH, CH)],
                             sem_z)
            return carry

        lax.fori_loop(0, NZ, zero_issue, 0)

        @pl.when(sub == 0)
        def _():
            pltpu.async_copy(zero_v.at[pl.ds(0, PAD)],
                             acc.at[pl.ds(RSZ, PAD)], sem_z)

        def zero_drain(k, carry):
            pltpu.make_async_copy(
                zero_v, acc.at[pl.ds(sub * ZPT + k * CH, CH)], sem_z).wait()
            return carry

        lax.fori_loop(0, NZ, zero_drain, 0)

        @pl.when(sub == 0)
        def _():
            pltpu.make_async_copy(zero_v.at[pl.ds(0, PAD)],
                                  acc.at[pl.ds(RSZ, PAD)], sem_z).wait()

        plsc.subcore_barrier()

        def chunk_pair(m, cnt):
            for s in range(2):
                g = m * 2 + s
                os = 1 - s

                @pl.when(g + 1 < NCH)
                def _():
                    load(g + 1, os)

                pltpu.make_async_copy(
                    idx_hbm.at[pl.ds(base_in + g * CH, CH)], idx_v[s],
                    sem_li[s]).wait()
                pltpu.make_async_copy(
                    feat_hbm.at[pl.ds(base_in + g * CH, CH)], feat_v[s],
                    sem_lf[s]).wait()

                def group(j, cnt):
                    o = j * 96
                    for k in range(6):
                        iv = idx_v[s][pl.ds(o + k * 16, 16)]
                        a = lax.shift_right_logical(iv, 5)
                        q = (a.astype(jnp.float32) * third).astype(jnp.int32)
                        off = q * 96 + cvec[k]
                        key = plsc.bitcast(off, jnp.uint32)
                        valid = key < jnp.uint32(RSZ)
                        fv = feat_v[s][pl.ds(o + k * 16, 16)]
                        fvz = jnp.where(valid, fv, jnp.float32(0.0))
                        skey, fvs = plsc.sort_key_val(key, fvz)
                        offs = plsc.bitcast(
                            jnp.minimum(skey, jnp.uint32(RSZ)), jnp.int32)
                        pend_off[pl.ds(cnt, 16)] = offs
                        pend_val[pl.ds(cnt, 16)] = fvs
                        cnt = cnt + plsc.all_reduce_population_count(valid)[0]
                    return maybe_flush(cnt)

                cnt = lax.fori_loop(0, NG, group, cnt)
            return cnt

        cnt = lax.fori_loop(0, NCH // 2, chunk_pair, jnp.int32(0))
        flush_body()
        plsc.subcore_barrier()

        out_base = b * POUT + rbase + sub * ZPT

        def write_issue(k, carry):
            pltpu.async_copy(acc.at[pl.ds(sub * ZPT + k * CH, CH)],
                             out_hbm.at[pl.ds(out_base + k * CH, CH)], sem_w)
            return carry

        lax.fori_loop(0, NZ, write_issue, 0)

        def write_drain(k, carry):
            pltpu.make_async_copy(
                acc.at[pl.ds(sub * ZPT + k * CH, CH)],
                out_hbm.at[pl.ds(out_base + k * CH, CH)], sem_w).wait()
            return carry

        lax.fori_loop(0, NZ, write_drain, 0)
        plsc.subcore_barrier()

    def batch_loop(b, carry):
        def region_loop(rr, c2):
            one_pass(b, rr)
            return c2

        lax.fori_loop(0, NREG // 2, region_loop, 0)
        return carry

    lax.fori_loop(0, B, batch_loop, 0)


def kernel(features, idxs):
    out_flat = _unpool(features.reshape(-1), idxs.reshape(-1))
    return out_flat.reshape(B, OUT_H, OUT_W, C)


# speedup vs baseline: 2.5790x; 2.5790x over previous
import functools

import jax
import jax.numpy as jnp
from jax import lax
from jax.experimental import pallas as pl
from jax.experimental.pallas import tpu as pltpu
from jax.experimental.pallas import tpu_sc as plsc

B, H, W, C = 4, 192, 192, 96
OUT_H, OUT_W = 2 * H, 2 * W
NUPD = H * W * C
POUT = OUT_H * OUT_W * C
NREG = 8
RSZ = POUT // NREG
PAD = 256
NSUB = 16
PER_TILE = NUPD // NSUB
CH = 3072
NCH = PER_TILE // CH
ZPT = RSZ // NSUB
NZ = ZPT // CH
NG = CH // 96

_mesh = plsc.VectorSubcoreMesh(core_axis_name="c", subcore_axis_name="s")


@functools.partial(
    pl.kernel,
    mesh=_mesh,
    out_type=jax.ShapeDtypeStruct((B * POUT,), jnp.float32),
    scratch_types=[
        [pltpu.VMEM((CH,), jnp.int32)] * 2,
        [pltpu.VMEM((CH,), jnp.float32)] * 2,
        pltpu.VMEM((CH,), jnp.int32),
        pltpu.VMEM((CH,), jnp.float32),
        pltpu.VMEM_SHARED((RSZ + PAD,), jnp.float32),
        [pltpu.SemaphoreType.DMA] * 2,
        [pltpu.SemaphoreType.DMA] * 2,
        pltpu.SemaphoreType.DMA,
        pltpu.SemaphoreType.DMA,
    ],
)
def _unpool(feat_hbm, idx_hbm, out_hbm, idx_v, feat_v, off_v, zero_v, acc,
            sem_li, sem_lf, sem_z, sem_w):
    core = lax.axis_index("c")
    sub = lax.axis_index("s")
    third = jnp.float32(1.0) / jnp.float32(3.0)
    iota = lax.iota(jnp.int32, 16)
    dump = jnp.int32(RSZ) + sub * 16 + iota

    def zfill(i, carry):
        zero_v[pl.ds(i * 16, 16)] = jnp.zeros((16,), jnp.float32)
        return carry

    lax.fori_loop(0, CH // 16, zfill, 0)

    def one_pass(b, rr):
        rbase = (core * (NREG // 2) + rr) * RSZ
        base_in = b * NUPD + sub * PER_TILE
        cvec = [iota + (16 * k - rbase) for k in range(6)]

        def load(g, slot):
            pltpu.async_copy(
                idx_hbm.at[pl.ds(base_in + g * CH, CH)], idx_v[slot],
                sem_li[slot])
            pltpu.async_copy(
                feat_hbm.at[pl.ds(base_in + g * CH, CH)], feat_v[slot],
                sem_lf[slot])

        load(0, 0)

        def zero_issue(k, carry):
            pltpu.async_copy(zero_v, acc.at[pl.ds(sub * ZPT + k * CH, CH)],
                             sem_z)
            return carry

        lax.fori_loop(0, NZ, zero_issue, 0)

        def zero_drain(k, carry):
            pltpu.make_async_copy(
                zero_v, acc.at[pl.ds(sub * ZPT + k * CH, CH)], sem_z).wait()
            return carry

        lax.fori_loop(0, NZ, zero_drain, 0)
        plsc.subcore_barrier()

        def chunk_pair(m, carry):
            for s in range(2):
                g = m * 2 + s
                os = 1 - s

                @pl.when(g + 1 < NCH)
                def _():
                    load(g + 1, os)

                pltpu.make_async_copy(
                    idx_hbm.at[pl.ds(base_in + g * CH, CH)], idx_v[s],
                    sem_li[s]).wait()
                pltpu.make_async_copy(
                    feat_hbm.at[pl.ds(base_in + g * CH, CH)], feat_v[s],
                    sem_lf[s]).wait()

                def group(j, c2):
                    o = j * 96
                    for k in range(6):
                        iv = idx_v[s][pl.ds(o + k * 16, 16)]
                        a = lax.shift_right_logical(iv, 5)
                        q = (a.astype(jnp.float32) * third).astype(jnp.int32)
                        off = q * 96 + cvec[k]
                        valid = plsc.bitcast(off, jnp.uint32) < jnp.uint32(RSZ)
                        off_v[pl.ds(o + k * 16, 16)] = jnp.where(
                            valid, off, dump)
                    return c2

                lax.fori_loop(0, NG, group, 0)
                pltpu.sync_copy(feat_v[s], acc.at[off_v], add=True)
            return carry

        lax.fori_loop(0, NCH // 2, chunk_pair, 0)
        plsc.subcore_barrier()

        out_base = b * POUT + rbase + sub * ZPT
        pltpu.async_copy(acc.at[pl.ds(sub * ZPT, ZPT)],
                         out_hbm.at[pl.ds(out_base, ZPT)], sem_w)
        pltpu.make_async_copy(acc.at[pl.ds(sub * ZPT, ZPT)],
                              out_hbm.at[pl.ds(out_base, ZPT)], sem_w).wait()
        plsc.subcore_barrier()

    def batch_loop(b, carry):
        def region_loop(rr, c2):
            one_pass(b, rr)
            return c2

        lax.fori_loop(0, NREG // 2, region_loop, 0)
        return carry

    lax.fori_loop(0, B, batch_loop, 0)


def kernel(features, idxs):
    out_flat = _unpool(features.reshape(-1), idxs.reshape(-1))
    return out_flat.reshape(B, OUT_H, OUT_W, C)
